# baseline (speedup 1.0000x reference)
import functools

import jax
import jax.numpy as jnp
from jax import lax
from jax.experimental import pallas as pl
from jax.experimental.pallas import tpu as pltpu
from jax.experimental.pallas import tpu_sc as plsc

_BATCH = 4096
_SEQ = 200
_D = 64
_NW = 32
_N = _BATCH * _SEQ
_SUB = 128
_NSUB = 4
_CHUNK = _SUB * _NSUB
_BCHUNKS = _BATCH // _CHUNK
_NTASK = _SEQ * _BCHUNKS
_TPW = _NTASK // _NW
_NBUF = 3
_LOOKAHEAD = 1
_LANES = 16
_VPR = _D // _LANES

_mesh = plsc.VectorSubcoreMesh(core_axis_name="c", subcore_axis_name="s")


@functools.partial(
    pl.kernel,
    out_type=jax.ShapeDtypeStruct((_BATCH, _SEQ, _D), jnp.float32),
    mesh=_mesh,
    scratch_types=[
        pltpu.VMEM((_SEQ, _D), jnp.float32),
        pltpu.VMEM((_NBUF, _NSUB, _SUB), jnp.int32),
        pltpu.VMEM((_NBUF, _CHUNK, _D), jnp.float32),
        pltpu.SemaphoreType.DMA((_NBUF,)),
        pltpu.SemaphoreType.DMA((_NBUF,)),
    ],
    compiler_params=pltpu.CompilerParams(use_tc_tiling_on_sc=False),
)
def _embed(idx_hbm, table_hbm, pos_hbm, out_hbm, pos_v, idx_v, rows_v,
           gsem, ssem):
    cid = lax.axis_index("c")
    sid = lax.axis_index("s")
    wid = sid * 2 + cid
    tbase = wid * _TPW

    def fire_gather(t, b):
        base = t * _CHUNK
        for k in range(_NSUB):
            pltpu.sync_copy(idx_hbm.at[pl.ds(base + k * _SUB, _SUB)],
                            idx_v.at[b, k])
        for k in range(_NSUB):
            pltpu.async_copy(table_hbm.at[idx_v.at[b, k]],
                             rows_v.at[b, pl.ds(k * _SUB, _SUB)],
                             gsem.at[b])

    def wait_gather(b):
        for k in range(_NSUB):
            pltpu.make_async_copy(
                table_hbm.at[idx_v.at[b, k]],
                rows_v.at[b, pl.ds(k * _SUB, _SUB)],
                gsem.at[b]).wait()

    def fire_scatter(t, b):
        s = t // _BCHUNKS
        b0 = (t % _BCHUNKS) * _CHUNK
        pltpu.async_copy(rows_v.at[b], out_hbm.at[pl.ds(b0, _CHUNK), s],
                         ssem.at[b])

    def wait_scatter(b):
        pltpu.make_async_copy(rows_v.at[b], out_hbm.at[pl.ds(0, _CHUNK), 0],
                              ssem.at[b]).wait()

    pltpu.sync_copy(pos_hbm, pos_v)
    for p in range(_LOOKAHEAD):
        fire_gather(tbase + p, p)

    @pl.loop(0, _TPW)
    def _task(c):
        t = tbase + c
        b = lax.rem(c, _NBUF)

        @pl.when(c + _LOOKAHEAD < _TPW)
        def _prefetch():
            nb = lax.rem(c + _LOOKAHEAD, _NBUF)

            @pl.when(c >= _NBUF - _LOOKAHEAD)
            def _drain():
                wait_scatter(nb)
            fire_gather(t + _LOOKAHEAD, nb)

        wait_gather(b)

        s = t // _BCHUNKS
        pv = [pos_v[s, pl.ds(v * _LANES, _LANES)] for v in range(_VPR)]

        @pl.loop(0, _CHUNK, unroll=8)
        def _row(r):
            for v in range(_VPR):
                plsc.addupdate(rows_v.at[b, r, pl.ds(v * _LANES, _LANES)],
                               pv[v])

        fire_scatter(t, b)

    for u in range(_NBUF):
        wait_scatter(jnp.int32((_TPW - 1 - u) % _NBUF))


def kernel(x, table, pos_enc):
    idx = x.T.reshape(_N)
    return _embed(idx, table, pos_enc)


# speedup vs baseline: 1.0022x; 1.0022x over previous
import functools

import jax
import jax.numpy as jnp
from jax import lax
from jax.experimental import pallas as pl
from jax.experimental.pallas import tpu as pltpu
from jax.experimental.pallas import tpu_sc as plsc

_BATCH = 4096
_SEQ = 200
_D = 64
_NW = 32
_N = _BATCH * _SEQ
_SUB = 128
_NSUB = 2
_CHUNK = _SUB * _NSUB
_BCHUNKS = _BATCH // _CHUNK
_NTASK = _SEQ * _BCHUNKS
_TPW = _NTASK // _NW
_NBUF = 4
_LOOKAHEAD = 2
_LANES = 16
_VPR = _D // _LANES

_mesh = plsc.VectorSubcoreMesh(core_axis_name="c", subcore_axis_name="s")


@functools.partial(
    pl.kernel,
    out_type=jax.ShapeDtypeStruct((_BATCH, _SEQ, _D), jnp.float32),
    mesh=_mesh,
    scratch_types=[
        pltpu.VMEM((_SEQ, _D), jnp.float32),
        pltpu.VMEM((_NBUF, _NSUB, _SUB), jnp.int32),
        pltpu.VMEM((_NBUF, _CHUNK, _D), jnp.float32),
        pltpu.SemaphoreType.DMA((_NBUF,)),
        pltpu.SemaphoreType.DMA((_NBUF,)),
    ],
    compiler_params=pltpu.CompilerParams(use_tc_tiling_on_sc=False),
)
def _embed(idx_hbm, table_hbm, pos_hbm, out_hbm, pos_v, idx_v, rows_v,
           gsem, ssem):
    cid = lax.axis_index("c")
    sid = lax.axis_index("s")
    wid = sid * 2 + cid
    tbase = wid * _TPW

    def fire_gather(t, b):
        base = t * _CHUNK
        for k in range(_NSUB):
            pltpu.sync_copy(idx_hbm.at[pl.ds(base + k * _SUB, _SUB)],
                            idx_v.at[b, k])
        for k in range(_NSUB):
            pltpu.async_copy(table_hbm.at[idx_v.at[b, k]],
                             rows_v.at[b, pl.ds(k * _SUB, _SUB)],
                             gsem.at[b])

    def wait_gather(b):
        for k in range(_NSUB):
            pltpu.make_async_copy(
                table_hbm.at[idx_v.at[b, k]],
                rows_v.at[b, pl.ds(k * _SUB, _SUB)],
                gsem.at[b]).wait()

    def fire_scatter(t, b):
        s = t // _BCHUNKS
        b0 = (t % _BCHUNKS) * _CHUNK
        pltpu.async_copy(rows_v.at[b], out_hbm.at[pl.ds(b0, _CHUNK), s],
                         ssem.at[b])

    def wait_scatter(b):
        pltpu.make_async_copy(rows_v.at[b], out_hbm.at[pl.ds(0, _CHUNK), 0],
                              ssem.at[b]).wait()

    pltpu.sync_copy(pos_hbm, pos_v)
    for p in range(_LOOKAHEAD):
        fire_gather(tbase + p, p)

    @pl.loop(0, _TPW)
    def _task(c):
        t = tbase + c
        b = lax.rem(c, _NBUF)

        @pl.when(c + _LOOKAHEAD < _TPW)
        def _prefetch():
            nb = lax.rem(c + _LOOKAHEAD, _NBUF)

            @pl.when(c >= _NBUF - _LOOKAHEAD)
            def _drain():
                wait_scatter(nb)
            fire_gather(t + _LOOKAHEAD, nb)

        wait_gather(b)

        s = t // _BCHUNKS
        pv = [pos_v[s, pl.ds(v * _LANES, _LANES)] for v in range(_VPR)]

        @pl.loop(0, _CHUNK, unroll=8)
        def _row(r):
            for v in range(_VPR):
                plsc.addupdate(rows_v.at[b, r, pl.ds(v * _LANES, _LANES)],
                               pv[v])

        fire_scatter(t, b)

    for u in range(_NBUF):
        wait_scatter(jnp.int32((_TPW - 1 - u) % _NBUF))


def kernel(x, table, pos_enc):
    idx = x.T.reshape(_N)
    return _embed(idx, table, pos_enc)
